# baseline (speedup 1.0000x reference)
import functools

import numpy as np

import jax
import jax.numpy as jnp
from jax.experimental import pallas as pl
from jax.experimental.pallas import tpu as pltpu
from jax.experimental.pallas import tpu_sc as plsc

N = 10000
E = 160000
NC, NS, L = 2, 16, 16
NPAD = 10240
EPAD = 163840
CHUNK = 128
NBLK = 16
NBUF = 6
NPW = NPAD // NS
EROWS = EPAD // CHUNK
DEG_CHUNKS = EPAD // (NC * NS * CHUNK)
SEG_CHUNKS = EPAD // (NS * CHUNK)
SEG_BLOCKS = SEG_CHUNKS // NBLK
RB = 1280

F1, FP1, FH1, NSP1 = 300, 384, 64, 6
F2, FP2, FH2, NSP2 = 100, 128, 64, 2
F3, FP3, FH3, NSP3 = 64, 64, 32, 2

_mesh = plsc.VectorSubcoreMesh(core_axis_name="c", subcore_axis_name="s",
                               num_cores=NC, num_subcores=NS)

_sc_params = pltpu.CompilerParams(needs_layout_passes=False,
                                  use_tc_tiling_on_sc=False)

_HI_MASK = np.int32(-65536)


def _deg_body(dst_hbm, ew_hbm, out_hbm, hist, dstv, ewv, tmpv, outv,
              sem0, sem1, accd):
    cidx = jax.lax.axis_index("c")
    sidx = jax.lax.axis_index("s")
    wid = cidx * NS + sidx

    d1 = pltpu.async_copy(dst_hbm.at[pl.ds(wid * DEG_CHUNKS, DEG_CHUNKS)],
                          dstv, sem0)
    d2 = pltpu.async_copy(ew_hbm.at[pl.ds(wid * DEG_CHUNKS, DEG_CHUNKS)],
                          ewv, sem1)

    @pl.loop(0, NPAD // L)
    def _(i):
        hist[pl.ds(i * L, L)] = jnp.zeros((L,), jnp.float32)

    d1.wait()
    d2.wait()

    @pl.loop(0, DEG_CHUNKS)
    def _(g):
        for k in range(CHUNK // L):
            sl = pl.ds(k * L, L)
            plsc.addupdate_scatter(hist, [dstv[g, sl]], ewv[g, sl])

    pltpu.sync_copy(hist, accd.at[sidx])
    plsc.subcore_barrier()
    pltpu.sync_copy(accd.at[:, pl.ds(sidx * NPW, NPW)], tmpv)
    for i in range(NPW // L):
        sl = pl.ds(i * L, L)
        outv[sl] = tmpv[0, sl]

    @pl.loop(1, NS)
    def _(t):
        for i in range(NPW // L):
            sl = pl.ds(i * L, L)
            outv[sl] = outv[sl] + tmpv[t, sl]

    pltpu.sync_copy(outv, out_hbm.at[cidx, pl.ds(sidx * NPW, NPW)])


def _deg_call(dst2, ew2):
    k = pl.kernel(
        _deg_body,
        out_type=jax.ShapeDtypeStruct((NC, NPAD), jnp.float32),
        mesh=_mesh,
        compiler_params=_sc_params,
        scratch_types=[
            pltpu.VMEM((NPAD,), jnp.float32),
            pltpu.VMEM((DEG_CHUNKS, CHUNK), jnp.int32),
            pltpu.VMEM((DEG_CHUNKS, CHUNK), jnp.float32),
            pltpu.VMEM((NS, NPW), jnp.float32),
            pltpu.VMEM((NPW,), jnp.float32),
            pltpu.SemaphoreType.DMA,
            pltpu.SemaphoreType.DMA,
            pltpu.VMEM_SHARED((NS, NPAD), jnp.float32),
        ],
    )
    return k(dst2, ew2)


def _scale_chunk(fh, braw, bb, frows, fb, ews, k):
    @pl.loop(0, CHUNK // L)
    def _(jb):
        ev = ews[k, pl.ds(jb * L, L)]
        for i in range(L):
            e = ev[i]
            j = jb * L + i
            for g in range(fh // 32):
                vi = braw[bb, j, pl.ds(g * L, L)]
                lo = plsc.bitcast(jax.lax.shift_left(vi, 16), jnp.float32)
                hi = plsc.bitcast(jax.lax.bitwise_and(vi, _HI_MASK),
                                  jnp.float32)
                frows[fb, j, pl.ds(g * 32, L)] = lo * e
                frows[fb, j, pl.ds(g * 32 + L, L)] = hi * e


def _seg_body(fh, nsplit, table, srcE, dstE, ewE, zrows, out_hbm,
              srcs, idxs, dsts, ews, braw, frows, gsems, ssems, isems, acc):
    cidx = jax.lax.axis_index("c")
    sidx = jax.lax.axis_index("s")

    @pl.loop(0, nsplit // NC)
    def _(p):
        q = p * NC + cidx
        off = q * NPAD
        pltpu.sync_copy(zrows, acc.at[pl.ds(sidx * NPW, NPW)])
        plsc.subcore_barrier()

        @pl.loop(0, SEG_BLOCKS)
        def _(blk):
            rowbase = sidx * SEG_CHUNKS + blk * NBLK
            i1 = pltpu.async_copy(srcE.at[pl.ds(rowbase, NBLK)], srcs,
                                  isems.at[0])
            i2 = pltpu.async_copy(dstE.at[pl.ds(rowbase, NBLK)], dsts,
                                  isems.at[1])
            i3 = pltpu.async_copy(ewE.at[pl.ds(rowbase, NBLK)], ews,
                                  isems.at[2])
            i1.wait()
            i2.wait()
            i3.wait()
            for k in range(NBLK):
                for j in range(CHUNK // L):
                    sl = pl.ds(j * L, L)
                    idxs[k, sl] = srcs[k, sl] + off
            gd = [None] * NBLK
            sd = [None] * NBLK
            for k in range(NBUF - 1):
                gd[k] = pltpu.async_copy(table.at[idxs.at[k]],
                                         braw.at[k], gsems.at[k])
            for k in range(NBLK):
                b = k % NBUF
                if k + NBUF - 1 < NBLK:
                    nb = (k + NBUF - 1) % NBUF
                    gd[k + NBUF - 1] = pltpu.async_copy(
                        table.at[idxs.at[k + NBUF - 1]], braw.at[nb],
                        gsems.at[nb])
                gd[k].wait()
                if k >= NBUF:
                    sd[k - NBUF].wait()
                _scale_chunk(fh, braw, b, frows, b, ews, k)
                sd[k] = pltpu.async_copy(frows.at[b], acc.at[dsts.at[k]],
                                         ssems.at[b], add=True)
            for t in range(max(0, NBLK - NBUF), NBLK):
                sd[t].wait()

        plsc.subcore_barrier()
        pltpu.sync_copy(acc.at[pl.ds(sidx * NPW, NPW)],
                        out_hbm.at[pl.ds(q * NPAD + sidx * NPW, NPW)])


def _seg_call(fh, nsplit, table, src2, dst2, ew2, zrows):
    k = pl.kernel(
        functools.partial(_seg_body, fh, nsplit),
        out_type=jax.ShapeDtypeStruct((nsplit * NPAD, fh), jnp.float32),
        mesh=_mesh,
        compiler_params=_sc_params,
        scratch_types=[
            pltpu.VMEM((NBLK, CHUNK), jnp.int32),
            pltpu.VMEM((NBLK, CHUNK), jnp.int32),
            pltpu.VMEM((NBLK, CHUNK), jnp.int32),
            pltpu.VMEM((NBLK, CHUNK), jnp.float32),
            pltpu.VMEM((NBUF, CHUNK, fh // 2), jnp.int32),
            pltpu.VMEM((NBUF, CHUNK, fh), jnp.float32),
            pltpu.SemaphoreType.DMA((NBUF,)),
            pltpu.SemaphoreType.DMA((NBUF,)),
            pltpu.SemaphoreType.DMA((3,)),
            pltpu.VMEM_SHARED((NPAD, fh), jnp.float32),
        ],
    )
    return k(table, src2, dst2, ew2, zrows)


def _pack_bf16_pairs(tab, ncols):
    def rnd(xi):
        lsb = jax.lax.bitwise_and(jax.lax.shift_right_logical(xi, 16), 1)
        return xi + 32767 + lsb

    parts = []
    for g in range(ncols // 32):
        a = tab[:, g * 32:g * 32 + L]
        b = tab[:, g * 32 + L:g * 32 + 32]
        ai = rnd(jax.lax.bitcast_convert_type(a, jnp.int32))
        bi = rnd(jax.lax.bitcast_convert_type(b, jnp.int32))
        parts.append(jax.lax.bitwise_or(
            jax.lax.shift_right_logical(ai, 16),
            jax.lax.bitwise_and(bi, _HI_MASK)))
    return jnp.concatenate(parts, axis=1)


def _mm1(xp, w1p, discol):
    def kern(a_ref, w_ref, d_ref, o_ref, ob_ref):
        xw = jnp.dot(a_ref[...], w_ref[...], preferred_element_type=jnp.float32)
        tab = d_ref[...] * xw
        tb = _pack_bf16_pairs(tab, NSP1 * FH1)
        for q in range(NSP1):
            o_ref[q] = tab[:, q * FH1:(q + 1) * FH1]
            ob_ref[q] = tb[:, q * (FH1 // 2):(q + 1) * (FH1 // 2)]

    return pl.pallas_call(
        kern,
        grid=(NPAD // RB,),
        in_specs=[pl.BlockSpec((RB, 256), lambda i: (i, 0)),
                  pl.BlockSpec((256, NSP1 * FH1), lambda i: (0, 0)),
                  pl.BlockSpec((RB, 1), lambda i: (i, 0))],
        out_specs=[pl.BlockSpec((NSP1, RB, FH1), lambda i: (0, i, 0)),
                   pl.BlockSpec((NSP1, RB, FH1 // 2), lambda i: (0, i, 0))],
        out_shape=[jax.ShapeDtypeStruct((NSP1, NPAD, FH1), jnp.float32),
                   jax.ShapeDtypeStruct((NSP1, NPAD, FH1 // 2), jnp.int32)],
    )(xp, w1p, discol)


def _combine_mm(seg, tab, discol, bp, wp, ns_in, fh_in, fp_in, ns_out, fh_out):
    def kern(s_ref, t_ref, d_ref, b_ref, w_ref, o_ref, ob_ref):
        segc = jnp.concatenate([s_ref[q] for q in range(ns_in)], axis=1)
        tabc = jnp.concatenate([t_ref[q] for q in range(ns_in)], axis=1)
        d = d_ref[...]
        h = jnp.maximum(d * (segc + tabc) + b_ref[...], 0.0)
        hw = jnp.dot(h, w_ref[...], preferred_element_type=jnp.float32)
        newtab = d * hw
        tb = _pack_bf16_pairs(newtab, ns_out * fh_out)
        for q in range(ns_out):
            o_ref[q] = newtab[:, q * fh_out:(q + 1) * fh_out]
            ob_ref[q] = tb[:, q * (fh_out // 2):(q + 1) * (fh_out // 2)]

    return pl.pallas_call(
        kern,
        grid=(NPAD // RB,),
        in_specs=[pl.BlockSpec((ns_in, RB, fh_in), lambda i: (0, i, 0)),
                  pl.BlockSpec((ns_in, RB, fh_in), lambda i: (0, i, 0)),
                  pl.BlockSpec((RB, 1), lambda i: (i, 0)),
                  pl.BlockSpec((1, fp_in), lambda i: (0, 0)),
                  pl.BlockSpec((fp_in, ns_out * fh_out), lambda i: (0, 0))],
        out_specs=[pl.BlockSpec((ns_out, RB, fh_out), lambda i: (0, i, 0)),
                   pl.BlockSpec((ns_out, RB, fh_out // 2), lambda i: (0, i, 0))],
        out_shape=[jax.ShapeDtypeStruct((ns_out, NPAD, fh_out), jnp.float32),
                   jax.ShapeDtypeStruct((ns_out, NPAD, fh_out // 2), jnp.int32)],
    )(seg, tab, discol, bp, wp)


def _final_mm(seg, tab, discol, b3p, fc1w, fc1b, fc2w, fc2b):
    def kern(s_ref, t_ref, d_ref, b_ref, w1_ref, b1_ref, w2_ref, b2_ref, o_ref):
        segc = jnp.concatenate([s_ref[q] for q in range(NSP3)], axis=1)
        tabc = jnp.concatenate([t_ref[q] for q in range(NSP3)], axis=1)
        h = jnp.maximum(d_ref[...] * (segc + tabc) + b_ref[...], 0.0)
        t = jnp.dot(h, w1_ref[...], preferred_element_type=jnp.float32) + b1_ref[...]
        o_ref[...] = jnp.dot(t, w2_ref[...], preferred_element_type=jnp.float32) + b2_ref[...]

    return pl.pallas_call(
        kern,
        grid=(NPAD // RB,),
        in_specs=[pl.BlockSpec((NSP3, RB, FH3), lambda i: (0, i, 0)),
                  pl.BlockSpec((NSP3, RB, FH3), lambda i: (0, i, 0)),
                  pl.BlockSpec((RB, 1), lambda i: (i, 0)),
                  pl.BlockSpec((1, FP3), lambda i: (0, 0)),
                  pl.BlockSpec((64, 16), lambda i: (0, 0)),
                  pl.BlockSpec((1, 16), lambda i: (0, 0)),
                  pl.BlockSpec((16, 1), lambda i: (0, 0)),
                  pl.BlockSpec((1, 1), lambda i: (0, 0))],
        out_specs=pl.BlockSpec((RB, 1), lambda i: (i, 0)),
        out_shape=jax.ShapeDtypeStruct((NPAD, 1), jnp.float32),
    )(seg, tab, discol, b3p, fc1w, fc1b, fc2w, fc2b)


def kernel(x, edge_index, edge_attr, W1, b1, W2, b2, W3, b3,
           fc1_W, fc1_b, fc2_W, fc2_b):
    src = edge_index[0].astype(jnp.int32)
    dst = edge_index[1].astype(jnp.int32)
    ew = edge_attr.astype(jnp.float32)
    src2 = jnp.pad(src, (0, EPAD - E)).reshape(EROWS, CHUNK)
    dst2 = jnp.pad(dst, (0, EPAD - E)).reshape(EROWS, CHUNK)
    ew2 = jnp.pad(ew, (0, EPAD - E)).reshape(EROWS, CHUNK)

    degA = _deg_call(dst2, ew2)
    deg = degA[0] + degA[1] + 1.0
    dis = jax.lax.rsqrt(deg)
    discol = dis.reshape(NPAD, 1)

    xp = jnp.pad(x, ((0, NPAD - N), (0, 0)))
    w1p = jnp.pad(W1, ((0, 0), (0, FP1 - F1)))
    b1p = jnp.pad(b1, (0, FP1 - F1)).reshape(1, FP1)
    w2p = jnp.pad(W2, ((0, FP1 - F1), (0, FP2 - F2)))
    b2p = jnp.pad(b2, (0, FP2 - F2)).reshape(1, FP2)
    w3p = jnp.pad(W3, ((0, FP2 - F2), (0, 0)))
    b3p = b3.reshape(1, FP3)
    z1 = jnp.zeros((NPW, FH1), jnp.float32)
    z2 = jnp.zeros((NPW, FH2), jnp.float32)
    z3 = jnp.zeros((NPW, FH3), jnp.float32)

    tab1, tab1b = _mm1(xp, w1p, discol)
    seg1 = _seg_call(FH1, NSP1, tab1b.reshape(NSP1 * NPAD, FH1 // 2),
                     src2, dst2, ew2, z1)
    seg1 = seg1.reshape(NSP1, NPAD, FH1)

    tab2, tab2b = _combine_mm(seg1, tab1, discol, b1p, w2p,
                              NSP1, FH1, FP1, NSP2, FH2)
    seg2 = _seg_call(FH2, NSP2, tab2b.reshape(NSP2 * NPAD, FH2 // 2),
                     src2, dst2, ew2, z2)
    seg2 = seg2.reshape(NSP2, NPAD, FH2)

    tab3, tab3b = _combine_mm(seg2, tab2, discol, b2p, w3p,
                              NSP2, FH2, FP2, NSP3, FH3)
    seg3 = _seg_call(FH3, NSP3, tab3b.reshape(NSP3 * NPAD, FH3 // 2),
                     src2, dst2, ew2, z3)
    seg3 = seg3.reshape(NSP3, NPAD, FH3)

    out = _final_mm(seg3, tab3, discol, b3p, fc1_W, fc1_b.reshape(1, 16),
                    fc2_W, fc2_b.reshape(1, 1))
    return out[:N]


# speedup vs baseline: 1.0023x; 1.0023x over previous
import functools

import numpy as np

import jax
import jax.numpy as jnp
from jax.experimental import pallas as pl
from jax.experimental.pallas import tpu as pltpu
from jax.experimental.pallas import tpu_sc as plsc

N = 10000
E = 160000
NC, NS, L = 2, 16, 16
NPAD = 10240
EPAD = 163840
CHUNK = 128
NBLK = 16
NBUF = 4
NPW = NPAD // NS
EROWS = EPAD // CHUNK
DEG_CHUNKS = EPAD // (NC * NS * CHUNK)
SEG_CHUNKS = EPAD // (NS * CHUNK)
SEG_BLOCKS = SEG_CHUNKS // NBLK
RB = 1280

F1, FP1, FH1, NSP1 = 300, 384, 64, 6
F2, FP2, FH2, NSP2 = 100, 128, 64, 2
F3, FP3, FH3, NSP3 = 64, 64, 32, 2

_mesh = plsc.VectorSubcoreMesh(core_axis_name="c", subcore_axis_name="s",
                               num_cores=NC, num_subcores=NS)

_sc_params = pltpu.CompilerParams(needs_layout_passes=False,
                                  use_tc_tiling_on_sc=False)

_HI_MASK = np.int32(-65536)


def _deg_body(dst_hbm, ew_hbm, out_hbm, hist, dstv, ewv, tmpv, outv,
              sem0, sem1, accd):
    cidx = jax.lax.axis_index("c")
    sidx = jax.lax.axis_index("s")
    wid = cidx * NS + sidx

    d1 = pltpu.async_copy(dst_hbm.at[pl.ds(wid * DEG_CHUNKS, DEG_CHUNKS)],
                          dstv, sem0)
    d2 = pltpu.async_copy(ew_hbm.at[pl.ds(wid * DEG_CHUNKS, DEG_CHUNKS)],
                          ewv, sem1)

    @pl.loop(0, NPAD // L)
    def _(i):
        hist[pl.ds(i * L, L)] = jnp.zeros((L,), jnp.float32)

    d1.wait()
    d2.wait()

    @pl.loop(0, DEG_CHUNKS)
    def _(g):
        for k in range(CHUNK // L):
            sl = pl.ds(k * L, L)
            plsc.addupdate_scatter(hist, [dstv[g, sl]], ewv[g, sl])

    pltpu.sync_copy(hist, accd.at[sidx])
    plsc.subcore_barrier()
    pltpu.sync_copy(accd.at[:, pl.ds(sidx * NPW, NPW)], tmpv)
    for i in range(NPW // L):
        sl = pl.ds(i * L, L)
        outv[sl] = tmpv[0, sl]

    @pl.loop(1, NS)
    def _(t):
        for i in range(NPW // L):
            sl = pl.ds(i * L, L)
            outv[sl] = outv[sl] + tmpv[t, sl]

    pltpu.sync_copy(outv, out_hbm.at[cidx, pl.ds(sidx * NPW, NPW)])


def _deg_call(dst2, ew2):
    k = pl.kernel(
        _deg_body,
        out_type=jax.ShapeDtypeStruct((NC, NPAD), jnp.float32),
        mesh=_mesh,
        compiler_params=_sc_params,
        scratch_types=[
            pltpu.VMEM((NPAD,), jnp.float32),
            pltpu.VMEM((DEG_CHUNKS, CHUNK), jnp.int32),
            pltpu.VMEM((DEG_CHUNKS, CHUNK), jnp.float32),
            pltpu.VMEM((NS, NPW), jnp.float32),
            pltpu.VMEM((NPW,), jnp.float32),
            pltpu.SemaphoreType.DMA,
            pltpu.SemaphoreType.DMA,
            pltpu.VMEM_SHARED((NS, NPAD), jnp.float32),
        ],
    )
    return k(dst2, ew2)


def _scale_chunk(fh, braw, bb, frows, fb, ews, k):
    @pl.loop(0, CHUNK // L)
    def _(jb):
        ev = ews[k, pl.ds(jb * L, L)]
        for i in range(L):
            e = ev[i]
            j = jb * L + i
            for g in range(fh // 32):
                vi = braw[bb, j, pl.ds(g * L, L)]
                lo = plsc.bitcast(jax.lax.shift_left(vi, 16), jnp.float32)
                hi = plsc.bitcast(jax.lax.bitwise_and(vi, _HI_MASK),
                                  jnp.float32)
                frows[fb, j, pl.ds(g * 32, L)] = lo * e
                frows[fb, j, pl.ds(g * 32 + L, L)] = hi * e


def _seg_body(fh, nsplit, table, srcE, dstE, ewE, zrows, out_hbm,
              srcs, idxs, dsts, ews, braw, frows, gsems, ssems, isems, acc):
    cidx = jax.lax.axis_index("c")
    sidx = jax.lax.axis_index("s")

    @pl.loop(0, nsplit // NC)
    def _(p):
        q = p * NC + cidx
        off = q * NPAD
        pltpu.sync_copy(zrows, acc.at[pl.ds(sidx * NPW, NPW)])
        plsc.subcore_barrier()

        @pl.loop(0, SEG_BLOCKS)
        def _(blk):
            rowbase = sidx * SEG_CHUNKS + blk * NBLK
            i1 = pltpu.async_copy(srcE.at[pl.ds(rowbase, NBLK)], srcs,
                                  isems.at[0])
            i2 = pltpu.async_copy(dstE.at[pl.ds(rowbase, NBLK)], dsts,
                                  isems.at[1])
            i3 = pltpu.async_copy(ewE.at[pl.ds(rowbase, NBLK)], ews,
                                  isems.at[2])
            i1.wait()
            i2.wait()
            i3.wait()
            for k in range(NBLK):
                for j in range(CHUNK // L):
                    sl = pl.ds(j * L, L)
                    idxs[k, sl] = srcs[k, sl] + off
            gd = [None] * NBLK
            sd = [None] * NBLK
            for k in range(NBUF - 1):
                gd[k] = pltpu.async_copy(table.at[idxs.at[k]],
                                         braw.at[k], gsems.at[k])
            for k in range(NBLK):
                b = k % NBUF
                if k + NBUF - 1 < NBLK:
                    nb = (k + NBUF - 1) % NBUF
                    gd[k + NBUF - 1] = pltpu.async_copy(
                        table.at[idxs.at[k + NBUF - 1]], braw.at[nb],
                        gsems.at[nb])
                gd[k].wait()
                if k >= NBUF:
                    sd[k - NBUF].wait()
                _scale_chunk(fh, braw, b, frows, b, ews, k)
                sd[k] = pltpu.async_copy(frows.at[b], acc.at[dsts.at[k]],
                                         ssems.at[b], add=True)
            for t in range(max(0, NBLK - NBUF), NBLK):
                sd[t].wait()

        plsc.subcore_barrier()
        pltpu.sync_copy(acc.at[pl.ds(sidx * NPW, NPW)],
                        out_hbm.at[pl.ds(q * NPAD + sidx * NPW, NPW)])


def _seg_call(fh, nsplit, table, src2, dst2, ew2, zrows):
    k = pl.kernel(
        functools.partial(_seg_body, fh, nsplit),
        out_type=jax.ShapeDtypeStruct((nsplit * NPAD, fh), jnp.float32),
        mesh=_mesh,
        compiler_params=_sc_params,
        scratch_types=[
            pltpu.VMEM((NBLK, CHUNK), jnp.int32),
            pltpu.VMEM((NBLK, CHUNK), jnp.int32),
            pltpu.VMEM((NBLK, CHUNK), jnp.int32),
            pltpu.VMEM((NBLK, CHUNK), jnp.float32),
            pltpu.VMEM((NBUF, CHUNK, fh // 2), jnp.int32),
            pltpu.VMEM((NBUF, CHUNK, fh), jnp.float32),
            pltpu.SemaphoreType.DMA((NBUF,)),
            pltpu.SemaphoreType.DMA((NBUF,)),
            pltpu.SemaphoreType.DMA((3,)),
            pltpu.VMEM_SHARED((NPAD, fh), jnp.float32),
        ],
    )
    return k(table, src2, dst2, ew2, zrows)


def _pack_bf16_pairs(tab, ncols):
    def rnd(xi):
        lsb = jax.lax.bitwise_and(jax.lax.shift_right_logical(xi, 16), 1)
        return xi + 32767 + lsb

    parts = []
    for g in range(ncols // 32):
        a = tab[:, g * 32:g * 32 + L]
        b = tab[:, g * 32 + L:g * 32 + 32]
        ai = rnd(jax.lax.bitcast_convert_type(a, jnp.int32))
        bi = rnd(jax.lax.bitcast_convert_type(b, jnp.int32))
        parts.append(jax.lax.bitwise_or(
            jax.lax.shift_right_logical(ai, 16),
            jax.lax.bitwise_and(bi, _HI_MASK)))
    return jnp.concatenate(parts, axis=1)


def _mm1(xp, w1p, discol):
    def kern(a_ref, w_ref, d_ref, o_ref, ob_ref):
        xw = jnp.dot(a_ref[...], w_ref[...], preferred_element_type=jnp.float32)
        tab = d_ref[...] * xw
        tb = _pack_bf16_pairs(tab, NSP1 * FH1)
        for q in range(NSP1):
            o_ref[q] = tab[:, q * FH1:(q + 1) * FH1]
            ob_ref[q] = tb[:, q * (FH1 // 2):(q + 1) * (FH1 // 2)]

    return pl.pallas_call(
        kern,
        grid=(NPAD // RB,),
        in_specs=[pl.BlockSpec((RB, 256), lambda i: (i, 0)),
                  pl.BlockSpec((256, NSP1 * FH1), lambda i: (0, 0)),
                  pl.BlockSpec((RB, 1), lambda i: (i, 0))],
        out_specs=[pl.BlockSpec((NSP1, RB, FH1), lambda i: (0, i, 0)),
                   pl.BlockSpec((NSP1, RB, FH1 // 2), lambda i: (0, i, 0))],
        out_shape=[jax.ShapeDtypeStruct((NSP1, NPAD, FH1), jnp.float32),
                   jax.ShapeDtypeStruct((NSP1, NPAD, FH1 // 2), jnp.int32)],
    )(xp, w1p, discol)


def _combine_mm(seg, tab, discol, bp, wp, ns_in, fh_in, fp_in, ns_out, fh_out):
    def kern(s_ref, t_ref, d_ref, b_ref, w_ref, o_ref, ob_ref):
        segc = jnp.concatenate([s_ref[q] for q in range(ns_in)], axis=1)
        tabc = jnp.concatenate([t_ref[q] for q in range(ns_in)], axis=1)
        d = d_ref[...]
        h = jnp.maximum(d * (segc + tabc) + b_ref[...], 0.0)
        hw = jnp.dot(h, w_ref[...], preferred_element_type=jnp.float32)
        newtab = d * hw
        tb = _pack_bf16_pairs(newtab, ns_out * fh_out)
        for q in range(ns_out):
            o_ref[q] = newtab[:, q * fh_out:(q + 1) * fh_out]
            ob_ref[q] = tb[:, q * (fh_out // 2):(q + 1) * (fh_out // 2)]

    return pl.pallas_call(
        kern,
        grid=(NPAD // RB,),
        in_specs=[pl.BlockSpec((ns_in, RB, fh_in), lambda i: (0, i, 0)),
                  pl.BlockSpec((ns_in, RB, fh_in), lambda i: (0, i, 0)),
                  pl.BlockSpec((RB, 1), lambda i: (i, 0)),
                  pl.BlockSpec((1, fp_in), lambda i: (0, 0)),
                  pl.BlockSpec((fp_in, ns_out * fh_out), lambda i: (0, 0))],
        out_specs=[pl.BlockSpec((ns_out, RB, fh_out), lambda i: (0, i, 0)),
                   pl.BlockSpec((ns_out, RB, fh_out // 2), lambda i: (0, i, 0))],
        out_shape=[jax.ShapeDtypeStruct((ns_out, NPAD, fh_out), jnp.float32),
                   jax.ShapeDtypeStruct((ns_out, NPAD, fh_out // 2), jnp.int32)],
    )(seg, tab, discol, bp, wp)


def _final_mm(seg, tab, discol, b3p, fc1w, fc1b, fc2w, fc2b):
    def kern(s_ref, t_ref, d_ref, b_ref, w1_ref, b1_ref, w2_ref, b2_ref, o_ref):
        segc = jnp.concatenate([s_ref[q] for q in range(NSP3)], axis=1)
        tabc = jnp.concatenate([t_ref[q] for q in range(NSP3)], axis=1)
        h = jnp.maximum(d_ref[...] * (segc + tabc) + b_ref[...], 0.0)
        t = jnp.dot(h, w1_ref[...], preferred_element_type=jnp.float32) + b1_ref[...]
        o_ref[...] = jnp.dot(t, w2_ref[...], preferred_element_type=jnp.float32) + b2_ref[...]

    return pl.pallas_call(
        kern,
        grid=(NPAD // RB,),
        in_specs=[pl.BlockSpec((NSP3, RB, FH3), lambda i: (0, i, 0)),
                  pl.BlockSpec((NSP3, RB, FH3), lambda i: (0, i, 0)),
                  pl.BlockSpec((RB, 1), lambda i: (i, 0)),
                  pl.BlockSpec((1, FP3), lambda i: (0, 0)),
                  pl.BlockSpec((64, 16), lambda i: (0, 0)),
                  pl.BlockSpec((1, 16), lambda i: (0, 0)),
                  pl.BlockSpec((16, 1), lambda i: (0, 0)),
                  pl.BlockSpec((1, 1), lambda i: (0, 0))],
        out_specs=pl.BlockSpec((RB, 1), lambda i: (i, 0)),
        out_shape=jax.ShapeDtypeStruct((NPAD, 1), jnp.float32),
    )(seg, tab, discol, b3p, fc1w, fc1b, fc2w, fc2b)


def kernel(x, edge_index, edge_attr, W1, b1, W2, b2, W3, b3,
           fc1_W, fc1_b, fc2_W, fc2_b):
    src = edge_index[0].astype(jnp.int32)
    dst = edge_index[1].astype(jnp.int32)
    ew = edge_attr.astype(jnp.float32)
    src2 = jnp.pad(src, (0, EPAD - E)).reshape(EROWS, CHUNK)
    dst2 = jnp.pad(dst, (0, EPAD - E)).reshape(EROWS, CHUNK)
    ew2 = jnp.pad(ew, (0, EPAD - E)).reshape(EROWS, CHUNK)

    degA = _deg_call(dst2, ew2)
    deg = degA[0] + degA[1] + 1.0
    dis = jax.lax.rsqrt(deg)
    discol = dis.reshape(NPAD, 1)

    xp = jnp.pad(x, ((0, NPAD - N), (0, 0)))
    w1p = jnp.pad(W1, ((0, 0), (0, FP1 - F1)))
    b1p = jnp.pad(b1, (0, FP1 - F1)).reshape(1, FP1)
    w2p = jnp.pad(W2, ((0, FP1 - F1), (0, FP2 - F2)))
    b2p = jnp.pad(b2, (0, FP2 - F2)).reshape(1, FP2)
    w3p = jnp.pad(W3, ((0, FP2 - F2), (0, 0)))
    b3p = b3.reshape(1, FP3)
    z1 = jnp.zeros((NPW, FH1), jnp.float32)
    z2 = jnp.zeros((NPW, FH2), jnp.float32)
    z3 = jnp.zeros((NPW, FH3), jnp.float32)

    tab1, tab1b = _mm1(xp, w1p, discol)
    seg1 = _seg_call(FH1, NSP1, tab1b.reshape(NSP1 * NPAD, FH1 // 2),
                     src2, dst2, ew2, z1)
    seg1 = seg1.reshape(NSP1, NPAD, FH1)

    tab2, tab2b = _combine_mm(seg1, tab1, discol, b1p, w2p,
                              NSP1, FH1, FP1, NSP2, FH2)
    seg2 = _seg_call(FH2, NSP2, tab2b.reshape(NSP2 * NPAD, FH2 // 2),
                     src2, dst2, ew2, z2)
    seg2 = seg2.reshape(NSP2, NPAD, FH2)

    tab3, tab3b = _combine_mm(seg2, tab2, discol, b2p, w3p,
                              NSP2, FH2, FP2, NSP3, FH3)
    seg3 = _seg_call(FH3, NSP3, tab3b.reshape(NSP3 * NPAD, FH3 // 2),
                     src2, dst2, ew2, z3)
    seg3 = seg3.reshape(NSP3, NPAD, FH3)

    out = _final_mm(seg3, tab3, discol, b3p, fc1_W, fc1_b.reshape(1, 16),
                    fc2_W, fc2_b.reshape(1, 1))
    return out[:N]
